# baseline (speedup 1.0000x reference)
import functools

import jax
import jax.numpy as jnp
from jax import lax
from jax.experimental import pallas as pl
from jax.experimental.pallas import tpu as pltpu
from jax.experimental.pallas import tpu_sc as plsc

ROWS = 8192
DIM = 128
NUM_CORES = 2
NUM_SUBCORES = 16
NUM_WORKERS = NUM_CORES * NUM_SUBCORES
ROWS_PER_WORKER = ROWS // NUM_WORKERS

_mesh = plsc.VectorSubcoreMesh(core_axis_name="c", subcore_axis_name="s")


@functools.partial(
    pl.kernel,
    mesh=_mesh,
    out_type=jax.ShapeDtypeStruct((ROWS, DIM), jnp.float32),
)
def _pos_embed_lookup(table_hbm, out_hbm):
    wid = lax.axis_index("s") * NUM_CORES + lax.axis_index("c")
    base = wid * ROWS_PER_WORKER
    pltpu.sync_copy(
        table_hbm.at[pl.ds(base, ROWS_PER_WORKER)],
        out_hbm.at[pl.ds(base, ROWS_PER_WORKER)],
    )


def kernel(x, pos_table):
    del x
    return _pos_embed_lookup(pos_table)


# speedup vs baseline: 6.3224x; 6.3224x over previous
import functools

import jax
import jax.numpy as jnp
from jax import lax
from jax.experimental import pallas as pl
from jax.experimental.pallas import tpu as pltpu
from jax.experimental.pallas import tpu_sc as plsc

ROWS = 8192
DIM = 128
NUM_CORES = 2
NUM_SUBCORES = 16
NUM_WORKERS = NUM_CORES * NUM_SUBCORES
ROWS_PER_WORKER = ROWS // NUM_WORKERS

_mesh = plsc.VectorSubcoreMesh(core_axis_name="c", subcore_axis_name="s")


NUM_CHUNKS = 4
CHUNK_ROWS = ROWS_PER_WORKER // NUM_CHUNKS


@functools.partial(
    pl.kernel,
    mesh=_mesh,
    out_type=jax.ShapeDtypeStruct((ROWS, DIM), jnp.float32),
    scratch_types=[
        pltpu.VMEM((2, CHUNK_ROWS, DIM), jnp.float32),
        pltpu.SemaphoreType.DMA,
        pltpu.SemaphoreType.DMA,
        pltpu.SemaphoreType.DMA,
        pltpu.SemaphoreType.DMA,
    ],
)
def _pos_embed_lookup(table_hbm, out_hbm, buf_v, rs0, rs1, ws0, ws1):
    wid = lax.axis_index("s") * NUM_CORES + lax.axis_index("c")
    base = wid * ROWS_PER_WORKER
    rsem = (rs0, rs1)
    wsem = (ws0, ws1)
    reads = [None] * NUM_CHUNKS
    writes = [None] * NUM_CHUNKS

    reads[0] = pltpu.async_copy(
        table_hbm.at[pl.ds(base, CHUNK_ROWS)], buf_v.at[0], rsem[0]
    )
    for i in range(NUM_CHUNKS):
        b = i % 2
        if i + 1 < NUM_CHUNKS:
            nb = (i + 1) % 2
            if i >= 1:
                writes[i - 1].wait()
            reads[i + 1] = pltpu.async_copy(
                table_hbm.at[pl.ds(base + (i + 1) * CHUNK_ROWS, CHUNK_ROWS)],
                buf_v.at[nb],
                rsem[nb],
            )
        reads[i].wait()
        writes[i] = pltpu.async_copy(
            buf_v.at[b],
            out_hbm.at[pl.ds(base + i * CHUNK_ROWS, CHUNK_ROWS)],
            wsem[b],
        )
    writes[NUM_CHUNKS - 2].wait()
    writes[NUM_CHUNKS - 1].wait()


def kernel(x, pos_table):
    del x
    return _pos_embed_lookup(pos_table)
